# baseline (speedup 1.0000x reference)
import functools

import jax
import jax.numpy as jnp
from jax import lax
from jax.experimental import pallas as pl
from jax.experimental.pallas import tpu as pltpu
from jax.experimental.pallas import tpu_sc as plsc

N = 10000
E = 320000
D = 128
HID = 128
LBL = 32
R = 16

NUM_SC = 2
NUM_TILES = 16
NW = NUM_SC * NUM_TILES
EPT = E // NW
CK = 80
NCH = EPT // CK
ZSTART = 624
CZ = 160


def _vgather(vals, idx16):
    dnums = lax.GatherDimensionNumbers(
        offset_dims=(), collapsed_slice_dims=(0,), start_index_map=(0,))
    return lax.gather(vals, idx16[:, None], dnums, (1,),
                      mode=lax.GatherScatterMode.PROMISE_IN_BOUNDS)


def _zero_f32(ref, rows, cols):
    def body(i, _):
        r = i // (cols // 16)
        col = (i % (cols // 16)) * 16
        ref[r, pl.ds(col, 16)] = jnp.zeros((16,), jnp.float32)
        return 0
    lax.fori_loop(0, rows * (cols // 16), body, 0)


def _sc_hist_body(dst_hbm, ty_hbm, cnt_hbm, dstb, tyb, cb, ones_v, zb, cnt_sp):
    core = lax.axis_index("c")
    sub = lax.axis_index("s")
    wid = core * NUM_TILES + sub

    _zero_f32(zb, 1, 2000)
    per = (R * N) // NUM_TILES
    for k in range(per // 2000):
        pltpu.sync_copy(zb.at[0], cnt_sp.at[pl.ds(sub * per + k * 2000, 2000)])
    def ob(i, _):
        ones_v[pl.ds(i * 16, 16)] = jnp.ones((16,), jnp.float32)
        return 0
    lax.fori_loop(0, CK // 16, ob, 0)
    plsc.subcore_barrier()

    def chunk(k, _):
        base = wid * EPT + k * CK
        pltpu.sync_copy(dst_hbm.at[pl.ds(base, CK)], dstb)
        pltpu.sync_copy(ty_hbm.at[pl.ds(base, CK)], tyb)
        def idx(j, _):
            d = dstb[pl.ds(j * 16, 16)]
            t = tyb[pl.ds(j * 16, 16)]
            cb[pl.ds(j * 16, 16)] = d * R + t
            return 0
        lax.fori_loop(0, CK // 16, idx, 0)
        pltpu.sync_copy(ones_v, cnt_sp.at[cb], add=True)
        return 0
    lax.fori_loop(0, NCH, chunk, 0)
    plsc.subcore_barrier()

    for k in range(per // 2000):
        pltpu.sync_copy(cnt_sp.at[pl.ds(sub * per + k * 2000, 2000)], zb.at[0])
        pltpu.sync_copy(zb.at[0],
                        cnt_hbm.at[pl.ds(core * (R * N) + sub * per + k * 2000, 2000)])


def _sc_hist(dst, ty):
    mesh = plsc.VectorSubcoreMesh(core_axis_name="c", subcore_axis_name="s")
    return pl.kernel(
        _sc_hist_body,
        out_type=jax.ShapeDtypeStruct((NUM_SC * R * N,), jnp.float32),
        mesh=mesh,
        scratch_types=[
            pltpu.VMEM((CK,), jnp.int32),
            pltpu.VMEM((CK,), jnp.int32),
            pltpu.VMEM((CK,), jnp.int32),
            pltpu.VMEM((CK,), jnp.float32),
            pltpu.VMEM((1, 2000), jnp.float32),
            pltpu.VMEM_SHARED((R * N,), jnp.float32),
        ],
    )(dst, ty)


def _sc_edge_body(W, tv_hbm, src_hbm, dst_hbm, ty_hbm, norm_hbm, agg_hbm,
                  srcb, tyb,
                  dstb0, gb0, cb0, nrmb0, rows0, sem0,
                  dstb1, gb1, cb1, nrmb1, rows1, sem1,
                  zb, agg_sp):
    core = lax.axis_index("c")
    sub = lax.axis_index("s")
    wid = core * NUM_TILES + sub

    slots = ((dstb0, gb0, cb0, nrmb0, rows0, sem0),
             (dstb1, gb1, cb1, nrmb1, rows1, sem1))

    _zero_f32(zb, CZ, W)
    for k in range(4):
        pltpu.sync_copy(zb, agg_sp.at[pl.ds(sub * ZSTART + k * CZ, CZ)])
    plsc.subcore_barrier()

    def fire(k, slot):
        dstb, gb, cb, nrmb, rows, sem = slot
        base = wid * EPT + k * CK
        pltpu.sync_copy(src_hbm.at[pl.ds(base, CK)], srcb)
        pltpu.sync_copy(dst_hbm.at[pl.ds(base, CK)], dstb)
        pltpu.sync_copy(ty_hbm.at[pl.ds(base, CK)], tyb)
        def idx(j, _):
            s = srcb[pl.ds(j * 16, 16)]
            d = dstb[pl.ds(j * 16, 16)]
            t = tyb[pl.ds(j * 16, 16)]
            gb[pl.ds(j * 16, 16)] = s * R + t
            cb[pl.ds(j * 16, 16)] = d * R + t
            return 0
        lax.fori_loop(0, CK // 16, idx, 0)
        pltpu.async_copy(norm_hbm.at[cb], nrmb, sem)
        pltpu.async_copy(tv_hbm.at[gb], rows, sem)

    def drain(slot):
        dstb, gb, cb, nrmb, rows, sem = slot
        pltpu.make_async_copy(norm_hbm.at[cb], nrmb, sem).wait()
        pltpu.make_async_copy(tv_hbm.at[gb], rows, sem).wait()
        for j in range(CK // 16):
            nv = nrmb[pl.ds(j * 16, 16)]
            def scale(i, _, nv=nv, j=j):
                sv = _vgather(nv, jnp.full((16,), i, jnp.int32))
                r = j * 16 + i
                for q in range(W // 16):
                    rows[r, pl.ds(q * 16, 16)] = rows[r, pl.ds(q * 16, 16)] * sv
                return 0
            lax.fori_loop(0, 16, scale, 0)
        pltpu.sync_copy(rows, agg_sp.at[dstb], add=True)

    fire(0, slots[0])
    def pair(p, _):
        k0 = 2 * p
        fire(k0 + 1, slots[1])
        drain(slots[0])
        fire(k0 + 2, slots[0])
        drain(slots[1])
        return 0
    lax.fori_loop(0, (NCH - 1) // 2, pair, 0)
    drain(slots[0])
    plsc.subcore_barrier()

    for k in range(4):
        sl = pl.ds(sub * ZSTART + k * CZ, CZ)
        pltpu.sync_copy(agg_sp.at[sl], zb)
        pltpu.sync_copy(zb, agg_hbm.at[core, sl])


def _sc_edge_pass(W, tv, src, dst, ty, norm):
    mesh = plsc.VectorSubcoreMesh(core_axis_name="c", subcore_axis_name="s")
    slot = [
        pltpu.VMEM((CK,), jnp.int32),
        pltpu.VMEM((CK,), jnp.int32),
        pltpu.VMEM((CK,), jnp.int32),
        pltpu.VMEM((CK,), jnp.float32),
        pltpu.VMEM((CK, W), jnp.float32),
        pltpu.SemaphoreType.DMA,
    ]
    return pl.kernel(
        functools.partial(_sc_edge_body, W),
        out_type=jax.ShapeDtypeStruct((NUM_SC, N, W), jnp.float32),
        mesh=mesh,
        scratch_types=[
            pltpu.VMEM((CK,), jnp.int32),
            pltpu.VMEM((CK,), jnp.int32),
        ] + slot + slot + [
            pltpu.VMEM((CZ, W), jnp.float32),
            pltpu.VMEM_SHARED((N, W), jnp.float32),
        ],
    )(tv, src, dst, ty, norm)


BN = 1000


def _tc_prologue_body(emb, wf, bf, w1cat, root1, b1, t1v, xr1b):
    x = jnp.dot(emb[0], wf[...], preferred_element_type=jnp.float32) + bf[...]
    t1v[...] = jnp.dot(x, w1cat[...], preferred_element_type=jnp.float32)
    xr1b[...] = jnp.dot(x, root1[...], preferred_element_type=jnp.float32) + b1[...]


def _tc_prologue(emb, wf, bf, w1cat, root1, b1):
    grid = (N // BN,)
    return pl.pallas_call(
        _tc_prologue_body,
        grid=grid,
        in_specs=[
            pl.BlockSpec((1, BN, D), lambda i: (0, i, 0)),
            pl.BlockSpec((D, D), lambda i: (0, 0)),
            pl.BlockSpec((1, D), lambda i: (0, 0)),
            pl.BlockSpec((D, R * HID), lambda i: (0, 0)),
            pl.BlockSpec((D, HID), lambda i: (0, 0)),
            pl.BlockSpec((1, HID), lambda i: (0, 0)),
        ],
        out_specs=[
            pl.BlockSpec((BN, R * HID), lambda i: (i, 0)),
            pl.BlockSpec((BN, HID), lambda i: (i, 0)),
        ],
        out_shape=[
            jax.ShapeDtypeStruct((N, R * HID), jnp.float32),
            jax.ShapeDtypeStruct((N, HID), jnp.float32),
        ],
    )(emb, wf, bf, w1cat, root1, b1)


def _tc_norm_body(cnt, norm):
    norm[...] = 1.0 / jnp.maximum(cnt[0] + cnt[1], 1.0)


def _tc_norm(cntp):
    cnt3 = cntp.reshape(NUM_SC, (R * N) // 128, 128)
    out = pl.pallas_call(
        _tc_norm_body,
        out_shape=jax.ShapeDtypeStruct(((R * N) // 128, 128), jnp.float32),
    )(cnt3)
    return out.reshape(R * N)


def _tc_mid_body(aggp, xr1b, w2cat, root2, b2, t2v, xr2b):
    out1 = jax.nn.relu(aggp[0] + aggp[1] + xr1b[...])
    t2v[...] = jnp.dot(out1, w2cat[...], preferred_element_type=jnp.float32)
    xr2b[...] = jnp.dot(out1, root2[...], preferred_element_type=jnp.float32) + b2[...]


def _tc_mid(aggp, xr1b, w2cat, root2, b2):
    grid = (N // BN,)
    return pl.pallas_call(
        _tc_mid_body,
        grid=grid,
        in_specs=[
            pl.BlockSpec((NUM_SC, BN, HID), lambda i: (0, i, 0)),
            pl.BlockSpec((BN, HID), lambda i: (i, 0)),
            pl.BlockSpec((HID, R * D), lambda i: (0, 0)),
            pl.BlockSpec((HID, LBL), lambda i: (0, 0)),
            pl.BlockSpec((1, LBL), lambda i: (0, 0)),
        ],
        out_specs=[
            pl.BlockSpec((BN, R * D), lambda i: (i, 0)),
            pl.BlockSpec((BN, LBL), lambda i: (i, 0)),
        ],
        out_shape=[
            jax.ShapeDtypeStruct((N, R * D), jnp.float32),
            jax.ShapeDtypeStruct((N, LBL), jnp.float32),
        ],
    )(aggp, xr1b, w2cat, root2, b2)


def _tc_final_body(aggp2, xr2b, out):
    out[...] = jax.nn.sigmoid(
        aggp2[0, :, :LBL] + aggp2[1, :, :LBL] + xr2b[...])


def _tc_final(aggp2, xr2b):
    return pl.pallas_call(
        _tc_final_body,
        out_shape=jax.ShapeDtypeStruct((N, LBL), jnp.float32),
    )(aggp2, xr2b)


def kernel(embedding, edge_index, edge_type, in_proj_w, in_proj_b,
           out_proj_w, out_proj_b, w1, root1, b1, w2, root2, b2):
    src = edge_index[0].astype(jnp.int32)
    dst = edge_index[1].astype(jnp.int32)
    ty = edge_type.astype(jnp.int32)

    wv = in_proj_w[2 * D:3 * D]
    bv = in_proj_b[2 * D:3 * D]
    wf = (out_proj_w @ wv).T
    bf = (bv @ out_proj_w.T + out_proj_b).reshape(1, D)

    w1cat = w1.transpose(1, 0, 2).reshape(D, R * HID)
    w2p = jnp.pad(w2, ((0, 0), (0, 0), (0, D - LBL)))
    w2cat = w2p.transpose(1, 0, 2).reshape(HID, R * D)

    cntp = _sc_hist(dst, ty)
    t1v, xr1b = _tc_prologue(embedding, wf, bf, w1cat, root1,
                             b1.reshape(1, HID))
    norm = _tc_norm(cntp)
    aggp = _sc_edge_pass(HID, t1v.reshape(N * R, HID), src, dst, ty, norm)
    t2v, xr2b = _tc_mid(aggp, xr1b, w2cat, root2, b2.reshape(1, LBL))
    aggp2 = _sc_edge_pass(D, t2v.reshape(N * R, D), src, dst, ty, norm)
    return _tc_final(aggp2, xr2b)


# speedup vs baseline: 1.0030x; 1.0030x over previous
import functools

import jax
import jax.numpy as jnp
from jax import lax
from jax.experimental import pallas as pl
from jax.experimental.pallas import tpu as pltpu
from jax.experimental.pallas import tpu_sc as plsc

N = 10000
E = 320000
D = 128
HID = 128
LBL = 32
R = 16

NUM_SC = 2
NUM_TILES = 16
NW = NUM_SC * NUM_TILES
EPT = E // NW
CK = 80
NCH = EPT // CK
ZSTART = 624
CZ = 160


def _vgather(vals, idx16):
    dnums = lax.GatherDimensionNumbers(
        offset_dims=(), collapsed_slice_dims=(0,), start_index_map=(0,))
    return lax.gather(vals, idx16[:, None], dnums, (1,),
                      mode=lax.GatherScatterMode.PROMISE_IN_BOUNDS)


def _zero_f32(ref, rows, cols):
    def body(i, _):
        r = i // (cols // 16)
        col = (i % (cols // 16)) * 16
        ref[r, pl.ds(col, 16)] = jnp.zeros((16,), jnp.float32)
        return 0
    lax.fori_loop(0, rows * (cols // 16), body, 0)


def _sc_hist_body(dst_hbm, ty_hbm, cnt_hbm, dstb, tyb, cb, ones_v, zb, cnt_sp):
    core = lax.axis_index("c")
    sub = lax.axis_index("s")
    wid = core * NUM_TILES + sub

    _zero_f32(zb, 1, 2000)
    per = (R * N) // NUM_TILES
    for k in range(per // 2000):
        pltpu.sync_copy(zb.at[0], cnt_sp.at[pl.ds(sub * per + k * 2000, 2000)])
    def ob(i, _):
        ones_v[pl.ds(i * 16, 16)] = jnp.ones((16,), jnp.float32)
        return 0
    lax.fori_loop(0, CK // 16, ob, 0)
    plsc.subcore_barrier()

    def chunk(k, _):
        base = wid * EPT + k * CK
        pltpu.sync_copy(dst_hbm.at[pl.ds(base, CK)], dstb)
        pltpu.sync_copy(ty_hbm.at[pl.ds(base, CK)], tyb)
        def idx(j, _):
            d = dstb[pl.ds(j * 16, 16)]
            t = tyb[pl.ds(j * 16, 16)]
            cb[pl.ds(j * 16, 16)] = d * R + t
            return 0
        lax.fori_loop(0, CK // 16, idx, 0)
        pltpu.sync_copy(ones_v, cnt_sp.at[cb], add=True)
        return 0
    lax.fori_loop(0, NCH, chunk, 0)
    plsc.subcore_barrier()

    for k in range(per // 2000):
        pltpu.sync_copy(cnt_sp.at[pl.ds(sub * per + k * 2000, 2000)], zb.at[0])
        pltpu.sync_copy(zb.at[0],
                        cnt_hbm.at[pl.ds(core * (R * N) + sub * per + k * 2000, 2000)])


def _sc_hist(dst, ty):
    mesh = plsc.VectorSubcoreMesh(core_axis_name="c", subcore_axis_name="s")
    return pl.kernel(
        _sc_hist_body,
        out_type=jax.ShapeDtypeStruct((NUM_SC * R * N,), jnp.float32),
        mesh=mesh,
        scratch_types=[
            pltpu.VMEM((CK,), jnp.int32),
            pltpu.VMEM((CK,), jnp.int32),
            pltpu.VMEM((CK,), jnp.int32),
            pltpu.VMEM((CK,), jnp.float32),
            pltpu.VMEM((1, 2000), jnp.float32),
            pltpu.VMEM_SHARED((R * N,), jnp.float32),
        ],
    )(dst, ty)


def _sc_edge_body(W, tv_hbm, src_hbm, dst_hbm, ty_hbm, norm_hbm, agg_hbm,
                  srcb, tyb,
                  dstb0, gb0, cb0, nrmb0, rows0, sem0,
                  dstb1, gb1, cb1, nrmb1, rows1, sem1,
                  zb, agg_sp):
    core = lax.axis_index("c")
    sub = lax.axis_index("s")
    wid = core * NUM_TILES + sub

    slots = ((dstb0, gb0, cb0, nrmb0, rows0, sem0),
             (dstb1, gb1, cb1, nrmb1, rows1, sem1))

    _zero_f32(zb, CZ, W)
    for k in range(4):
        pltpu.sync_copy(zb, agg_sp.at[pl.ds(sub * ZSTART + k * CZ, CZ)])
    plsc.subcore_barrier()

    def fire(k, slot):
        dstb, gb, cb, nrmb, rows, sem = slot
        base = wid * EPT + k * CK
        pltpu.sync_copy(src_hbm.at[pl.ds(base, CK)], srcb)
        pltpu.sync_copy(dst_hbm.at[pl.ds(base, CK)], dstb)
        pltpu.sync_copy(ty_hbm.at[pl.ds(base, CK)], tyb)
        def idx(j, _):
            s = srcb[pl.ds(j * 16, 16)]
            d = dstb[pl.ds(j * 16, 16)]
            t = tyb[pl.ds(j * 16, 16)]
            gb[pl.ds(j * 16, 16)] = s * R + t
            cb[pl.ds(j * 16, 16)] = d * R + t
            return 0
        lax.fori_loop(0, CK // 16, idx, 0)
        pltpu.async_copy(norm_hbm.at[cb], nrmb, sem)
        pltpu.async_copy(tv_hbm.at[gb], rows, sem)

    def drain(slot):
        dstb, gb, cb, nrmb, rows, sem = slot
        pltpu.make_async_copy(norm_hbm.at[cb], nrmb, sem).wait()
        pltpu.make_async_copy(tv_hbm.at[gb], rows, sem).wait()
        for j in range(CK // 16):
            nv = nrmb[pl.ds(j * 16, 16)]
            def scale(i, _, nv=nv, j=j):
                sv = _vgather(nv, jnp.full((16,), i, jnp.int32))
                r = j * 16 + i
                for q in range(W // 16):
                    rows[r, pl.ds(q * 16, 16)] = rows[r, pl.ds(q * 16, 16)] * sv
                return 0
            lax.fori_loop(0, 16, scale, 0)
        pltpu.sync_copy(rows, agg_sp.at[dstb], add=True)

    fire(0, slots[0])
    def pair(p, _):
        k0 = 2 * p
        fire(k0 + 1, slots[1])
        drain(slots[0])
        fire(k0 + 2, slots[0])
        drain(slots[1])
        return 0
    lax.fori_loop(0, (NCH - 1) // 2, pair, 0)
    drain(slots[0])
    plsc.subcore_barrier()

    for k in range(4):
        sl = pl.ds(sub * ZSTART + k * CZ, CZ)
        pltpu.sync_copy(agg_sp.at[sl], zb)
        pltpu.sync_copy(zb, agg_hbm.at[core, sl])


def _sc_edge_pass(W, tv, src, dst, ty, norm):
    mesh = plsc.VectorSubcoreMesh(core_axis_name="c", subcore_axis_name="s")
    slot = [
        pltpu.VMEM((CK,), jnp.int32),
        pltpu.VMEM((CK,), jnp.int32),
        pltpu.VMEM((CK,), jnp.int32),
        pltpu.VMEM((CK,), jnp.float32),
        pltpu.VMEM((CK, W), jnp.float32),
        pltpu.SemaphoreType.DMA,
    ]
    return pl.kernel(
        functools.partial(_sc_edge_body, W),
        out_type=jax.ShapeDtypeStruct((NUM_SC, N, W), jnp.float32),
        mesh=mesh,
        scratch_types=[
            pltpu.VMEM((CK,), jnp.int32),
            pltpu.VMEM((CK,), jnp.int32),
        ] + slot + slot + [
            pltpu.VMEM((CZ, W), jnp.float32),
            pltpu.VMEM_SHARED((N, W), jnp.float32),
        ],
    )(tv, src, dst, ty, norm)


BN = 1000


def _tc_prologue_body(emb, wf, bf, w1cat, root1, b1, t1v, xr1b):
    x = jnp.dot(emb[0], wf[...], preferred_element_type=jnp.float32) + bf[...]
    t1v[...] = jnp.dot(x, w1cat[...], preferred_element_type=jnp.float32)
    xr1b[...] = jnp.dot(x, root1[...], preferred_element_type=jnp.float32) + b1[...]


def _tc_prologue(emb, wf, bf, w1cat, root1, b1):
    grid = (N // BN,)
    return pl.pallas_call(
        _tc_prologue_body,
        grid=grid,
        in_specs=[
            pl.BlockSpec((1, BN, D), lambda i: (0, i, 0)),
            pl.BlockSpec((D, D), lambda i: (0, 0)),
            pl.BlockSpec((1, D), lambda i: (0, 0)),
            pl.BlockSpec((D, R * HID), lambda i: (0, 0)),
            pl.BlockSpec((D, HID), lambda i: (0, 0)),
            pl.BlockSpec((1, HID), lambda i: (0, 0)),
        ],
        out_specs=[
            pl.BlockSpec((BN, R * HID), lambda i: (i, 0)),
            pl.BlockSpec((BN, HID), lambda i: (i, 0)),
        ],
        out_shape=[
            jax.ShapeDtypeStruct((N, R * HID), jnp.float32),
            jax.ShapeDtypeStruct((N, HID), jnp.float32),
        ],
    )(emb, wf, bf, w1cat, root1, b1)


def _tc_norm_body(cnt, norm):
    norm[...] = 1.0 / jnp.maximum(cnt[0] + cnt[1], 1.0)


def _tc_norm(cntp):
    cnt3 = cntp.reshape(NUM_SC, (R * N) // 128, 128)
    out = pl.pallas_call(
        _tc_norm_body,
        out_shape=jax.ShapeDtypeStruct(((R * N) // 128, 128), jnp.float32),
    )(cnt3)
    return out.reshape(R * N)


def _tc_mid_body(aggp, xr1b, w2cat, root2, b2, t2v, xr2b):
    out1 = jax.nn.relu(aggp[0] + aggp[1] + xr1b[...])
    t2v[...] = jnp.dot(out1, w2cat[...], preferred_element_type=jnp.float32)
    xr2b[...] = jnp.dot(out1, root2[...], preferred_element_type=jnp.float32) + b2[...]


def _tc_mid(aggp, xr1b, w2cat, root2, b2):
    grid = (N // BN,)
    return pl.pallas_call(
        _tc_mid_body,
        grid=grid,
        in_specs=[
            pl.BlockSpec((NUM_SC, BN, HID), lambda i: (0, i, 0)),
            pl.BlockSpec((BN, HID), lambda i: (i, 0)),
            pl.BlockSpec((HID, R * D), lambda i: (0, 0)),
            pl.BlockSpec((HID, LBL), lambda i: (0, 0)),
            pl.BlockSpec((1, LBL), lambda i: (0, 0)),
        ],
        out_specs=[
            pl.BlockSpec((BN, R * D), lambda i: (i, 0)),
            pl.BlockSpec((BN, LBL), lambda i: (i, 0)),
        ],
        out_shape=[
            jax.ShapeDtypeStruct((N, R * D), jnp.float32),
            jax.ShapeDtypeStruct((N, LBL), jnp.float32),
        ],
    )(aggp, xr1b, w2cat, root2, b2)


def _tc_final_body(aggp2, xr2b, out):
    out[...] = jax.nn.sigmoid(
        aggp2[0, :, :LBL] + aggp2[1, :, :LBL] + xr2b[...])


def _tc_final(aggp2, xr2b):
    return pl.pallas_call(
        _tc_final_body,
        out_shape=jax.ShapeDtypeStruct((N, LBL), jnp.float32),
    )(aggp2, xr2b)


def kernel(embedding, edge_index, edge_type, in_proj_w, in_proj_b,
           out_proj_w, out_proj_b, w1, root1, b1, w2, root2, b2):
    src = edge_index[0].astype(jnp.int32)
    dst = edge_index[1].astype(jnp.int32)
    ty = edge_type.astype(jnp.int32)

    wv = in_proj_w[2 * D:3 * D]
    bv = in_proj_b[2 * D:3 * D]
    wf = (out_proj_w @ wv).T
    bf = (bv @ out_proj_w.T + out_proj_b).reshape(1, D)

    w1cat = w1.transpose(1, 0, 2).reshape(D, R * HID)
    w2p = jnp.pad(w2, ((0, 0), (0, 0), (0, D - LBL)))
    w2cat = w2p.transpose(1, 0, 2).reshape(HID, R * D)

    t1v, xr1b = _tc_prologue(embedding, wf, bf, w1cat, root1,
                             b1.reshape(1, HID))
    cntp = _sc_hist(dst, ty)
    norm = _tc_norm(cntp)
    aggp = _sc_edge_pass(HID, t1v.reshape(N * R, HID), src, dst, ty, norm)
    t2v, xr2b = _tc_mid(aggp, xr1b, w2cat, root2, b2.reshape(1, LBL))
    aggp2 = _sc_edge_pass(D, t2v.reshape(N * R, D), src, dst, ty, norm)
    return _tc_final(aggp2, xr2b)
